# baseline (speedup 1.0000x reference)
import jax
import jax.numpy as jnp
from jax.experimental import pallas as pl
from jax.experimental.pallas import tpu as pltpu


def _body(pos_ref, prev_ref, cur_ref, out_ref):
    out_ref[...] = prev_ref[...]
    p = pos_ref[0]
    out_ref[:, :, pl.ds(p, 1), :] = cur_ref[...]


def kernel(prev, cur, dim, idx):
    B1, B2, S, D = prev.shape
    pos = (idx - (dim - 1)).astype(jnp.int32)
    BH = 4
    grid = (B1, B2 // BH)
    out = pl.pallas_call(
        _body,
        grid_spec=pltpu.PrefetchScalarGridSpec(
            num_scalar_prefetch=1,
            grid=grid,
            in_specs=[
                pl.BlockSpec((1, BH, S, D), lambda i, j, p: (i, j, 0, 0)),
                pl.BlockSpec((1, BH, 1, D), lambda i, j, p: (i, j, 0, 0)),
            ],
            out_specs=pl.BlockSpec((1, BH, S, D), lambda i, j, p: (i, j, 0, 0)),
        ),
        out_shape=jax.ShapeDtypeStruct(prev.shape, prev.dtype),
        compiler_params=pltpu.CompilerParams(
            dimension_semantics=("parallel", "parallel"),
        ),
    )(pos, prev, cur)
    return out
